# Initial kernel scaffold; baseline (speedup 1.0000x reference)
import jax
import jax.numpy as jnp
from jax.experimental import pallas as pl


def kernel(inputs, edge_index, edge_weight, W0, b0, W1, b1, W2, b2, Wh1, bh1, Wh2, bh2, Wh3, bh3):
    raise NotImplementedError("write your pallas kernel here")


# speedup vs baseline: 2.7250x; 2.7250x over previous
import functools

import jax
import jax.numpy as jnp
from jax import lax
from jax.experimental import pallas as pl
from jax.experimental.pallas import tpu as pltpu
from jax.experimental.pallas import tpu_sc as plsc

N = 10000
E = 160000
B = 2
F = 128
NSUB = 16
EPS = E // NSUB
NPS = N // NSUB
NPAD = 10240
NPSA = NPAD // NSUB
CHD = 80
NCHD = EPS // CHD
NCH = EPS // 16

_mesh = plsc.VectorSubcoreMesh(core_axis_name="c", subcore_axis_name="s")
_sc_params = pltpu.CompilerParams(needs_layout_passes=False,
                                  use_tc_tiling_on_sc=False)


def _newton_rsqrt(d):
    ibits = lax.bitcast_convert_type(d, jnp.int32)
    h = jnp.int32(0x5F3759DF) - (ibits >> 1)
    y = lax.bitcast_convert_type(h, jnp.float32)
    for _ in range(4):
        y = y * (1.5 - 0.5 * d * y * y)
    return jnp.where(d > 0, y, 0.0)


@functools.partial(
    pl.kernel,
    out_type=jax.ShapeDtypeStruct((E,), jnp.float32),
    mesh=_mesh,
    compiler_params=_sc_params,
    scratch_types=[
        pltpu.VMEM((NCHD, CHD), jnp.int32),
        pltpu.VMEM((EPS,), jnp.int32),
        pltpu.VMEM((EPS,), jnp.float32),
        pltpu.VMEM((NPSA,), jnp.float32),
        pltpu.VMEM((NPSA,), jnp.float32),
        pltpu.VMEM((NPAD,), jnp.float32),
        pltpu.VMEM((EPS,), jnp.float32),
        pltpu.VMEM_SHARED((NPAD,), jnp.float32),
        pltpu.VMEM_SHARED((NPAD,), jnp.float32),
    ],
)
def _edge_vals(rowdeg_hbm, zeros_hbm, col_hbm, w_hbm,
               val_hbm, rowdegv, colv, wv, dv, disv640, disv,
               valv, degacc, dissh):
    c = lax.axis_index("c")
    s = lax.axis_index("s")

    @pl.when(c == 0)
    def _():
        pltpu.sync_copy(zeros_hbm.at[pl.ds(s * NPSA, NPSA)],
                        degacc.at[pl.ds(s * NPSA, NPSA)])
        pltpu.sync_copy(rowdeg_hbm.at[s], rowdegv)
        pltpu.sync_copy(w_hbm.at[pl.ds(s * EPS, EPS)], wv)
        plsc.subcore_barrier()

        def degbody(i, carry):
            pltpu.sync_copy(wv.at[pl.ds(i * CHD, CHD)],
                            degacc.at[rowdegv.at[i]], add=True)
            return carry
        lax.fori_loop(0, NCHD, degbody, 0)
        plsc.subcore_barrier()

        pltpu.sync_copy(degacc.at[pl.ds(s * NPSA, NPSA)], dv)

        def rsbody(k, carry):
            d = dv[pl.ds(k * 16, 16)]
            disv640[pl.ds(k * 16, 16)] = _newton_rsqrt(d)
            return carry
        lax.fori_loop(0, NPSA // 16, rsbody, 0)
        pltpu.sync_copy(disv640, dissh.at[pl.ds(s * NPSA, NPSA)])
        plsc.subcore_barrier()
        pltpu.sync_copy(dissh, disv)

        pltpu.sync_copy(col_hbm.at[pl.ds(s * EPS, EPS)], colv)

        def valbody(r, carry):
            for j in range(CHD // 16):
                base = r * CHD + j * 16
                r16 = rowdegv[r, pl.ds(j * 16, 16)]
                c16 = colv[pl.ds(base, 16)]
                w16 = wv[pl.ds(base, 16)]
                dr = plsc.load_gather(disv, [r16])
                dc = plsc.load_gather(disv, [c16])
                valv[pl.ds(base, 16)] = (-2.0) * w16 * dr * dc
            return carry
        lax.fori_loop(0, NCHD, valbody, 0)
        pltpu.sync_copy(valv, val_hbm.at[pl.ds(s * EPS, EPS)])


FH = F // 2


@functools.partial(
    pl.kernel,
    out_type=(jax.ShapeDtypeStruct((B * NPAD, FH), jnp.float32),
              jax.ShapeDtypeStruct((B * NPAD, FH), jnp.float32)),
    mesh=_mesh,
    compiler_params=_sc_params,
    scratch_types=[
        pltpu.VMEM((NCH, 16), jnp.int32),
        pltpu.VMEM((EPS,), jnp.int32),
        pltpu.VMEM((EPS,), jnp.float32),
        pltpu.VMEM((16, FH), jnp.float32),
        pltpu.VMEM((32, FH), jnp.float32),
        pltpu.VMEM_SHARED((NPAD, FH), jnp.float32),
        pltpu.SemaphoreType.DMA,
    ],
)
def _spmm(xh_hbm, row3d_hbm, col_hbm, val_hbm, out0_hbm, out1_hbm,
          rowv, colv, valv, gbuf, zbuf, acc, sem):
    c = lax.axis_index("c")
    s = lax.axis_index("s")
    pltpu.sync_copy(row3d_hbm.at[s], rowv)
    pltpu.sync_copy(col_hbm.at[pl.ds(s * EPS, EPS)], colv)
    pltpu.sync_copy(val_hbm.at[pl.ds(s * EPS, EPS)], valv)

    for r in range(32):
        for j in range(FH // 16):
            zbuf[r, pl.ds(j * 16, 16)] = jnp.zeros((16,), jnp.float32)

    coff = c * N
    zero16 = jnp.zeros((16,), jnp.int32)

    for h, out_hbm in ((0, out0_hbm), (1, out1_hbm)):
        def zbody(t, carry):
            pltpu.sync_copy(zbuf, acc.at[pl.ds(s * NPSA + t * 32, 32)])
            return carry
        lax.fori_loop(0, NPSA // 32, zbody, 0)
        plsc.subcore_barrier()

        def chunk(i, carry):
            g16 = (colv[pl.ds(i * 16, 16)] + coff) * 2 + h
            pltpu.async_copy(xh_hbm.at[g16], gbuf, sem).wait()
            for e in range(16):
                vb = plsc.load_gather(valv, [i * 16 + e + zero16])
                for j in range(FH // 16):
                    gbuf[e, pl.ds(j * 16, 16)] = (
                        gbuf[e, pl.ds(j * 16, 16)] * vb)
            pltpu.sync_copy(gbuf, acc.at[rowv.at[i]], add=True)
            return carry
        lax.fori_loop(0, NCH, chunk, 0)
        plsc.subcore_barrier()
        pltpu.sync_copy(acc.at[pl.ds(s * NPSA, NPSA)],
                        out_hbm.at[pl.ds(c * NPAD + s * NPSA, NPSA)])
        plsc.subcore_barrier()


def _dense_body(x_ref, p0_ref, p1_ref, w_ref, b_ref, o_ref):
    w = w_ref[...]
    wb = w[1] + 2.0 * w[2] + w[3]
    wa = w[0] - w[2] - 0.1 * wb
    acc = jnp.dot(x_ref[0], wa, preferred_element_type=jnp.float32)
    acc = acc + jnp.dot(p0_ref[0], wb[:FH], preferred_element_type=jnp.float32)
    acc = acc + jnp.dot(p1_ref[0], wb[FH:], preferred_element_type=jnp.float32)
    acc = acc + b_ref[...]
    o_ref[0] = jnp.maximum(acc, 0.0)


_BN = 2000

_dense = pl.pallas_call(
    _dense_body,
    grid=(B, N // _BN),
    in_specs=[
        pl.BlockSpec((1, _BN, F), lambda b, i: (b, i, 0)),
        pl.BlockSpec((1, _BN, FH), lambda b, i: (b, i, 0)),
        pl.BlockSpec((1, _BN, FH), lambda b, i: (b, i, 0)),
        pl.BlockSpec((4, F, F), lambda b, i: (0, 0, 0)),
        pl.BlockSpec((1, F), lambda b, i: (0, 0)),
    ],
    out_specs=pl.BlockSpec((1, _BN, F), lambda b, i: (b, i, 0)),
    out_shape=jax.ShapeDtypeStruct((B, N, F), jnp.float32),
)


def _head_body(x3_ref, w1_ref, b1_ref, w2_ref, b2_ref, w3_ref, b3_ref, o_ref):
    pooled = jnp.max(x3_ref[...], axis=2)
    h = jnp.dot(pooled, w1_ref[...], preferred_element_type=jnp.float32)
    h = jnp.maximum(h + b1_ref[...], 0.0)
    h = jnp.dot(h, w2_ref[...], preferred_element_type=jnp.float32)
    h = jnp.maximum(h + b2_ref[...], 0.0)
    o_ref[...] = jnp.dot(h, w3_ref[...],
                         preferred_element_type=jnp.float32) + b3_ref[...]


def _head(x3, w1, b1, w2, b2, w3, b3):
    return pl.pallas_call(
        _head_body,
        out_shape=jax.ShapeDtypeStruct((B, 10), jnp.float32),
    )(x3, w1, b1, w2, b2, w3, b3)


def kernel(inputs, edge_index, edge_weight, W0, b0, W1, b1, W2, b2,
           Wh1, bh1, Wh2, bh2, Wh3, bh3):
    row = edge_index[0]
    col = edge_index[1]
    rowdeg = row.reshape(NSUB, NCHD, CHD)
    row3d = row.reshape(NSUB, NCH, 16)
    zerosn = jnp.zeros((NPAD,), jnp.float32)

    val = _edge_vals(rowdeg, zerosn, col, edge_weight)

    x = inputs
    for W, b in ((W0, b0), (W1, b1), (W2, b2)):
        pf0, pf1 = _spmm(x.reshape(B * N * 2, FH), row3d, col, val)
        p0 = pf0.reshape(B, NPAD, FH)[:, :N]
        p1 = pf1.reshape(B, NPAD, FH)[:, :N]
        x = _dense(x, p0, p1, W, b.reshape(1, F))
    return _head(x, Wh1, bh1.reshape(1, F), Wh2, bh2.reshape(1, 32),
                 Wh3, bh3.reshape(1, 10))


# speedup vs baseline: 6.2527x; 2.2946x over previous
import functools

import jax
import jax.numpy as jnp
from jax import lax
from jax.experimental import pallas as pl
from jax.experimental.pallas import tpu as pltpu
from jax.experimental.pallas import tpu_sc as plsc

N = 10000
E = 160000
B = 2
F = 128
FH = F // 2
NSUB = 16
EPS = E // NSUB
CH = 80
NCHB = 126
EPSP = NCHB * CH
EPAD = NSUB * EPSP
NPAD = 10240
NPSA = NPAD // NSUB

_mesh = plsc.VectorSubcoreMesh(core_axis_name="c", subcore_axis_name="s")
_sc_params = pltpu.CompilerParams(needs_layout_passes=False,
                                  use_tc_tiling_on_sc=False)


def _newton_rsqrt(d):
    ibits = lax.bitcast_convert_type(d, jnp.int32)
    h = jnp.int32(0x5F3759DF) - (ibits >> 1)
    y = lax.bitcast_convert_type(h, jnp.float32)
    for _ in range(4):
        y = y * (1.5 - 0.5 * d * y * y)
    return jnp.where(d > 0, y, 0.0)


@functools.partial(
    pl.kernel,
    out_type=jax.ShapeDtypeStruct((EPAD,), jnp.float32),
    mesh=_mesh,
    compiler_params=_sc_params,
    scratch_types=[
        pltpu.VMEM((NCHB, CH), jnp.int32),
        pltpu.VMEM((EPSP,), jnp.int32),
        pltpu.VMEM((EPSP,), jnp.float32),
        pltpu.VMEM((NPSA,), jnp.float32),
        pltpu.VMEM((NPSA,), jnp.float32),
        pltpu.VMEM((NPAD,), jnp.float32),
        pltpu.VMEM((EPSP,), jnp.float32),
        pltpu.VMEM_SHARED((NPAD,), jnp.float32),
        pltpu.VMEM_SHARED((NPAD,), jnp.float32),
    ],
)
def _edge_vals(row3d_hbm, zeros_hbm, col_hbm, w_hbm,
               val_hbm, rowv, colv, wv, dv, disv640, disv,
               valv, degacc, dissh):
    c = lax.axis_index("c")
    s = lax.axis_index("s")

    @pl.when(c == 0)
    def _():
        pltpu.sync_copy(zeros_hbm.at[pl.ds(s * NPSA, NPSA)],
                        degacc.at[pl.ds(s * NPSA, NPSA)])
        pltpu.sync_copy(row3d_hbm.at[s], rowv)
        pltpu.sync_copy(w_hbm.at[pl.ds(s * EPSP, EPSP)], wv)
        plsc.subcore_barrier()

        def degbody(i, carry):
            pltpu.sync_copy(wv.at[pl.ds(i * CH, CH)],
                            degacc.at[rowv.at[i]], add=True)
            return carry
        lax.fori_loop(0, NCHB, degbody, 0)
        plsc.subcore_barrier()

        pltpu.sync_copy(degacc.at[pl.ds(s * NPSA, NPSA)], dv)

        def rsbody(k, carry):
            d = dv[pl.ds(k * 16, 16)]
            disv640[pl.ds(k * 16, 16)] = _newton_rsqrt(d)
            return carry
        lax.fori_loop(0, NPSA // 16, rsbody, 0)
        pltpu.sync_copy(disv640, dissh.at[pl.ds(s * NPSA, NPSA)])
        plsc.subcore_barrier()
        pltpu.sync_copy(dissh, disv)

        pltpu.sync_copy(col_hbm.at[pl.ds(s * EPSP, EPSP)], colv)

        def valbody(r, carry):
            for j in range(CH // 16):
                base = r * CH + j * 16
                r16 = rowv[r, pl.ds(j * 16, 16)]
                c16 = colv[pl.ds(base, 16)]
                w16 = wv[pl.ds(base, 16)]
                dr = plsc.load_gather(disv, [r16])
                dc = plsc.load_gather(disv, [c16])
                valv[pl.ds(base, 16)] = (-2.0) * w16 * dr * dc
            return carry
        lax.fori_loop(0, NCHB, valbody, 0)
        pltpu.sync_copy(valv, val_hbm.at[pl.ds(s * EPSP, EPSP)])


@functools.partial(
    pl.kernel,
    out_type=(jax.ShapeDtypeStruct((B * NPAD, FH), jnp.float32),
              jax.ShapeDtypeStruct((B * NPAD, FH), jnp.float32)),
    mesh=_mesh,
    compiler_params=_sc_params,
    scratch_types=[
        pltpu.VMEM((NCHB, CH), jnp.int32),
        pltpu.VMEM((EPSP,), jnp.int32),
        pltpu.VMEM((EPSP,), jnp.float32),
        pltpu.VMEM((EPSP,), jnp.int32),
        pltpu.VMEM((CH, FH), jnp.float32),
        pltpu.VMEM((CH, FH), jnp.float32),
        pltpu.VMEM((32, FH), jnp.float32),
        pltpu.VMEM_SHARED((NPAD, FH), jnp.float32),
        pltpu.SemaphoreType.DMA,
        pltpu.SemaphoreType.DMA,
    ],
)
def _spmm(xh_hbm, row3d_hbm, col_hbm, val_hbm, out0_hbm, out1_hbm,
          rowv, colv, valv, gidx, gb0, gb1, zbuf, acc, sem0, sem1):
    c = lax.axis_index("c")
    s = lax.axis_index("s")
    pltpu.sync_copy(row3d_hbm.at[s], rowv)
    pltpu.sync_copy(col_hbm.at[pl.ds(s * EPSP, EPSP)], colv)
    pltpu.sync_copy(val_hbm.at[pl.ds(s * EPSP, EPSP)], valv)

    for r in range(32):
        for j in range(FH // 16):
            zbuf[r, pl.ds(j * 16, 16)] = jnp.zeros((16,), jnp.float32)

    coff2 = c * (2 * N)
    zero16 = jnp.zeros((16,), jnp.int32)

    def issue(i, gb, sem):
        pltpu.async_copy(xh_hbm.at[gidx.at[pl.ds(i * CH, CH)]], gb, sem)

    def wait(gb, sem):
        pltpu.make_async_copy(xh_hbm.at[pl.ds(0, CH)], gb, sem).wait()

    def scale(i, gb):
        for e in range(CH):
            vb = plsc.load_gather(valv, [i * CH + e + zero16])
            for j in range(FH // 16):
                gb[e, pl.ds(j * 16, 16)] = gb[e, pl.ds(j * 16, 16)] * vb

    for h, out_hbm in ((0, out0_hbm), (1, out1_hbm)):
        def gixbody(k, carry):
            gidx[pl.ds(k * 16, 16)] = (
                colv[pl.ds(k * 16, 16)] * 2 + (coff2 + h))
            return carry
        lax.fori_loop(0, EPSP // 16, gixbody, 0)

        def zbody(t, carry):
            pltpu.sync_copy(zbuf, acc.at[pl.ds(s * NPSA + t * 32, 32)])
            return carry
        lax.fori_loop(0, NPSA // 32, zbody, 0)
        plsc.subcore_barrier()

        issue(0, gb0, sem0)

        def pair(g, carry):
            i0 = 2 * g
            i1 = 2 * g + 1
            issue(i1, gb1, sem1)
            wait(gb0, sem0)
            scale(i0, gb0)
            pltpu.sync_copy(gb0, acc.at[rowv.at[i0]], add=True)

            @pl.when(g < NCHB // 2 - 1)
            def _():
                issue(i0 + 2, gb0, sem0)
            wait(gb1, sem1)
            scale(i1, gb1)
            pltpu.sync_copy(gb1, acc.at[rowv.at[i1]], add=True)
            return carry
        lax.fori_loop(0, NCHB // 2, pair, 0)
        plsc.subcore_barrier()
        pltpu.sync_copy(acc.at[pl.ds(s * NPSA, NPSA)],
                        out_hbm.at[pl.ds(c * NPAD + s * NPSA, NPSA)])
        plsc.subcore_barrier()


def _dense_body(x_ref, p0_ref, p1_ref, w_ref, b_ref, o_ref):
    w = w_ref[...]
    wb = w[1] + 2.0 * w[2] + w[3]
    wa = w[0] - w[2] - 0.1 * wb
    acc = jnp.dot(x_ref[0], wa, preferred_element_type=jnp.float32)
    acc = acc + jnp.dot(p0_ref[0], wb[:FH], preferred_element_type=jnp.float32)
    acc = acc + jnp.dot(p1_ref[0], wb[FH:], preferred_element_type=jnp.float32)
    acc = acc + b_ref[...]
    o_ref[0] = jnp.maximum(acc, 0.0)


_BN = 2000

_dense = pl.pallas_call(
    _dense_body,
    grid=(B, N // _BN),
    in_specs=[
        pl.BlockSpec((1, _BN, F), lambda b, i: (b, i, 0)),
        pl.BlockSpec((1, _BN, FH), lambda b, i: (b, i, 0)),
        pl.BlockSpec((1, _BN, FH), lambda b, i: (b, i, 0)),
        pl.BlockSpec((4, F, F), lambda b, i: (0, 0, 0)),
        pl.BlockSpec((1, F), lambda b, i: (0, 0)),
    ],
    out_specs=pl.BlockSpec((1, _BN, F), lambda b, i: (b, i, 0)),
    out_shape=jax.ShapeDtypeStruct((B, N, F), jnp.float32),
)


def _head_body(x3_ref, w1_ref, b1_ref, w2_ref, b2_ref, w3_ref, b3_ref, o_ref):
    pooled = jnp.max(x3_ref[...], axis=2)
    h = jnp.dot(pooled, w1_ref[...], preferred_element_type=jnp.float32)
    h = jnp.maximum(h + b1_ref[...], 0.0)
    h = jnp.dot(h, w2_ref[...], preferred_element_type=jnp.float32)
    h = jnp.maximum(h + b2_ref[...], 0.0)
    o_ref[...] = jnp.dot(h, w3_ref[...],
                         preferred_element_type=jnp.float32) + b3_ref[...]


def _head(x3, w1, b1, w2, b2, w3, b3):
    return pl.pallas_call(
        _head_body,
        out_shape=jax.ShapeDtypeStruct((B, 10), jnp.float32),
    )(x3, w1, b1, w2, b2, w3, b3)


def kernel(inputs, edge_index, edge_weight, W0, b0, W1, b1, W2, b2,
           Wh1, bh1, Wh2, bh2, Wh3, bh3):
    padi = jnp.zeros((NSUB, CH), jnp.int32)
    padf = jnp.zeros((NSUB, CH), jnp.float32)
    row_p = jnp.concatenate(
        [edge_index[0].reshape(NSUB, EPS), padi], axis=1)
    row3d = row_p.reshape(NSUB, NCHB, CH)
    col_p = jnp.concatenate(
        [edge_index[1].reshape(NSUB, EPS), padi], axis=1).reshape(EPAD)
    w_p = jnp.concatenate(
        [edge_weight.reshape(NSUB, EPS), padf], axis=1).reshape(EPAD)
    zerosn = jnp.zeros((NPAD,), jnp.float32)

    val = _edge_vals(row3d, zerosn, col_p, w_p)

    x = inputs
    for W, b in ((W0, b0), (W1, b1), (W2, b2)):
        pf0, pf1 = _spmm(x.reshape(B * N * 2, FH), row3d, col_p, val)
        p0 = pf0.reshape(B, NPAD, FH)[:, :N]
        p1 = pf1.reshape(B, NPAD, FH)[:, :N]
        x = _dense(x, p0, p1, W, b.reshape(1, F))
    return _head(x, Wh1, bh1.reshape(1, F), Wh2, bh2.reshape(1, 32),
                 Wh3, bh3.reshape(1, 10))


# speedup vs baseline: 10.1496x; 1.6232x over previous
import functools

import jax
import jax.numpy as jnp
from jax import lax
from jax.experimental import pallas as pl
from jax.experimental.pallas import tpu as pltpu
from jax.experimental.pallas import tpu_sc as plsc

N = 10000
E = 160000
B = 2
F = 128
FH = F // 2
NSUB = 16
EPS = E // NSUB
CH = 80
NCHB = 126
EPSP = NCHB * CH
EPAD = NSUB * EPSP
NPAD = 10240
NPSA = NPAD // NSUB

_mesh = plsc.VectorSubcoreMesh(core_axis_name="c", subcore_axis_name="s")
_sc_params = pltpu.CompilerParams(needs_layout_passes=False,
                                  use_tc_tiling_on_sc=False)


def _newton_rsqrt(d):
    ibits = lax.bitcast_convert_type(d, jnp.int32)
    h = jnp.int32(0x5F3759DF) - (ibits >> 1)
    y = lax.bitcast_convert_type(h, jnp.float32)
    for _ in range(4):
        y = y * (1.5 - 0.5 * d * y * y)
    return jnp.where(d > 0, y, 0.0)


@functools.partial(
    pl.kernel,
    out_type=jax.ShapeDtypeStruct((EPAD,), jnp.float32),
    mesh=_mesh,
    compiler_params=_sc_params,
    scratch_types=[
        pltpu.VMEM((NCHB, CH), jnp.int32),
        pltpu.VMEM((EPSP,), jnp.int32),
        pltpu.VMEM((EPSP,), jnp.float32),
        pltpu.VMEM((NPSA,), jnp.float32),
        pltpu.VMEM((NPSA,), jnp.float32),
        pltpu.VMEM((NPAD,), jnp.float32),
        pltpu.VMEM((EPSP,), jnp.float32),
        pltpu.VMEM_SHARED((NPAD,), jnp.float32),
        pltpu.VMEM_SHARED((NPAD,), jnp.float32),
    ],
)
def _edge_vals(row3d_hbm, zeros_hbm, col_hbm, w_hbm,
               val_hbm, rowv, colv, wv, dv, disv640, disv,
               valv, degacc, dissh):
    c = lax.axis_index("c")
    s = lax.axis_index("s")

    @pl.when(c == 0)
    def _():
        pltpu.sync_copy(zeros_hbm.at[pl.ds(s * NPSA, NPSA)],
                        degacc.at[pl.ds(s * NPSA, NPSA)])
        pltpu.sync_copy(row3d_hbm.at[s], rowv)
        pltpu.sync_copy(w_hbm.at[pl.ds(s * EPSP, EPSP)], wv)
        plsc.subcore_barrier()

        def degbody(i, carry):
            pltpu.sync_copy(wv.at[pl.ds(i * CH, CH)],
                            degacc.at[rowv.at[i]], add=True)
            return carry
        lax.fori_loop(0, NCHB, degbody, 0)
        plsc.subcore_barrier()

        pltpu.sync_copy(degacc.at[pl.ds(s * NPSA, NPSA)], dv)

        def rsbody(k, carry):
            d = dv[pl.ds(k * 16, 16)]
            disv640[pl.ds(k * 16, 16)] = _newton_rsqrt(d)
            return carry
        lax.fori_loop(0, NPSA // 16, rsbody, 0)
        pltpu.sync_copy(disv640, dissh.at[pl.ds(s * NPSA, NPSA)])
        plsc.subcore_barrier()
        pltpu.sync_copy(dissh, disv)

        pltpu.sync_copy(col_hbm.at[pl.ds(s * EPSP, EPSP)], colv)

        def valbody(r, carry):
            for j in range(CH // 16):
                base = r * CH + j * 16
                r16 = rowv[r, pl.ds(j * 16, 16)]
                c16 = colv[pl.ds(base, 16)]
                w16 = wv[pl.ds(base, 16)]
                dr = plsc.load_gather(disv, [r16])
                dc = plsc.load_gather(disv, [c16])
                valv[pl.ds(base, 16)] = (-2.0) * w16 * dr * dc
            return carry
        lax.fori_loop(0, NCHB, valbody, 0)
        pltpu.sync_copy(valv, val_hbm.at[pl.ds(s * EPSP, EPSP)])


@functools.partial(
    pl.kernel,
    out_type=jax.ShapeDtypeStruct((2, B * NPAD, FH), jnp.float32),
    mesh=_mesh,
    compiler_params=_sc_params,
    scratch_types=[
        pltpu.VMEM((NCHB, CH), jnp.int32),
        pltpu.VMEM((EPSP,), jnp.int32),
        pltpu.VMEM((EPSP,), jnp.float32),
        pltpu.VMEM((EPSP,), jnp.int32),
        pltpu.VMEM((CH, FH), jnp.float32),
        pltpu.VMEM((CH, FH), jnp.float32),
        pltpu.VMEM((CH, FH), jnp.float32),
        pltpu.VMEM((32, FH), jnp.float32),
        pltpu.VMEM_SHARED((NPAD, FH), jnp.float32),
        pltpu.SemaphoreType.DMA,
        pltpu.SemaphoreType.DMA,
        pltpu.SemaphoreType.DMA,
        pltpu.SemaphoreType.DMA,
        pltpu.SemaphoreType.DMA,
        pltpu.SemaphoreType.DMA,
    ],
)
def _spmm(xh_hbm, row3d_hbm, col_hbm, val_hbm, out_hbm,
          rowv, colv, valv, gidx, gb0, gb1, gb2, zbuf, acc,
          gs0, gs1, gs2, ss0, ss1, ss2):
    c = lax.axis_index("c")
    s = lax.axis_index("s")
    pltpu.sync_copy(row3d_hbm.at[s], rowv)
    pltpu.sync_copy(col_hbm.at[pl.ds(s * EPSP, EPSP)], colv)
    pltpu.sync_copy(val_hbm.at[pl.ds(s * EPSP, EPSP)], valv)

    for r in range(32):
        for j in range(FH // 16):
            zbuf[r, pl.ds(j * 16, 16)] = jnp.zeros((16,), jnp.float32)

    coff2 = c * (2 * N)
    bufs = (gb0, gb1, gb2)
    gsems = (gs0, gs1, gs2)
    ssems = (ss0, ss1, ss2)

    def issue(i, gb, sem):
        pltpu.async_copy(xh_hbm.at[gidx.at[pl.ds(i * CH, CH)]], gb, sem)

    def wait(gb, sem):
        pltpu.make_async_copy(xh_hbm.at[pl.ds(0, CH)], gb, sem).wait()

    def scale(i, gb):
        for g16 in range(CH // 16):
            v16 = valv[pl.ds(i * CH + g16 * 16, 16)]
            for e16 in range(16):
                e = g16 * 16 + e16
                vb = jnp.broadcast_to(v16[e16], (16,))
                for j in range(FH // 16):
                    gb[e, pl.ds(j * 16, 16)] = gb[e, pl.ds(j * 16, 16)] * vb

    def half(h, carry):
        def gixbody(k, kcarry):
            gidx[pl.ds(k * 16, 16)] = (
                colv[pl.ds(k * 16, 16)] * 2 + (coff2 + h))
            return kcarry
        lax.fori_loop(0, EPSP // 16, gixbody, 0)

        def zbody(t, tcarry):
            pltpu.sync_copy(zbuf, acc.at[pl.ds(s * NPSA + t * 32, 32)])
            return tcarry
        lax.fori_loop(0, NPSA // 32, zbody, 0)
        plsc.subcore_barrier()

        issue(0, gb0, gs0)

        def tri(g, tcarry):
            for q in range(3):
                j = 3 * g + q
                nq = (q + 1) % 3

                @pl.when(j >= 2)
                def _():
                    wait(bufs[nq], ssems[nq])

                @pl.when(j < NCHB - 1)
                def _():
                    issue(j + 1, bufs[nq], gsems[nq])
                wait(bufs[q], gsems[q])
                scale(j, bufs[q])
                pltpu.async_copy(bufs[q], acc.at[rowv.at[j]],
                                 ssems[q], add=True)
            return tcarry
        lax.fori_loop(0, NCHB // 3, tri, 0)
        wait(bufs[1], ssems[1])
        wait(bufs[2], ssems[2])
        plsc.subcore_barrier()
        pltpu.sync_copy(
            acc.at[pl.ds(s * NPSA, NPSA)],
            out_hbm.at[h].at[pl.ds(c * NPAD + s * NPSA, NPSA)])
        plsc.subcore_barrier()
        return carry
    lax.fori_loop(0, 2, half, 0)


def _dense_body(x_ref, p0_ref, p1_ref, w_ref, b_ref, o_ref):
    w = w_ref[...]
    wb = w[1] + 2.0 * w[2] + w[3]
    wa = w[0] - w[2] - 0.1 * wb
    acc = jnp.dot(x_ref[0], wa, preferred_element_type=jnp.float32)
    acc = acc + jnp.dot(p0_ref[0], wb[:FH], preferred_element_type=jnp.float32)
    acc = acc + jnp.dot(p1_ref[0], wb[FH:], preferred_element_type=jnp.float32)
    acc = acc + b_ref[...]
    o_ref[0] = jnp.maximum(acc, 0.0)


_BN = 2000

_dense = pl.pallas_call(
    _dense_body,
    grid=(B, N // _BN),
    in_specs=[
        pl.BlockSpec((1, _BN, F), lambda b, i: (b, i, 0)),
        pl.BlockSpec((1, _BN, FH), lambda b, i: (b, i, 0)),
        pl.BlockSpec((1, _BN, FH), lambda b, i: (b, i, 0)),
        pl.BlockSpec((4, F, F), lambda b, i: (0, 0, 0)),
        pl.BlockSpec((1, F), lambda b, i: (0, 0)),
    ],
    out_specs=pl.BlockSpec((1, _BN, F), lambda b, i: (b, i, 0)),
    out_shape=jax.ShapeDtypeStruct((B, N, F), jnp.float32),
)


def _head_body(x3_ref, w1_ref, b1_ref, w2_ref, b2_ref, w3_ref, b3_ref, o_ref):
    pooled = jnp.max(x3_ref[...], axis=2)
    h = jnp.dot(pooled, w1_ref[...], preferred_element_type=jnp.float32)
    h = jnp.maximum(h + b1_ref[...], 0.0)
    h = jnp.dot(h, w2_ref[...], preferred_element_type=jnp.float32)
    h = jnp.maximum(h + b2_ref[...], 0.0)
    o_ref[...] = jnp.dot(h, w3_ref[...],
                         preferred_element_type=jnp.float32) + b3_ref[...]


def _head(x3, w1, b1, w2, b2, w3, b3):
    return pl.pallas_call(
        _head_body,
        out_shape=jax.ShapeDtypeStruct((B, 10), jnp.float32),
    )(x3, w1, b1, w2, b2, w3, b3)


def kernel(inputs, edge_index, edge_weight, W0, b0, W1, b1, W2, b2,
           Wh1, bh1, Wh2, bh2, Wh3, bh3):
    padi = jnp.zeros((NSUB, CH), jnp.int32)
    padf = jnp.zeros((NSUB, CH), jnp.float32)
    row_p = jnp.concatenate(
        [edge_index[0].reshape(NSUB, EPS), padi], axis=1)
    row3d = row_p.reshape(NSUB, NCHB, CH)
    col_p = jnp.concatenate(
        [edge_index[1].reshape(NSUB, EPS), padi], axis=1).reshape(EPAD)
    w_p = jnp.concatenate(
        [edge_weight.reshape(NSUB, EPS), padf], axis=1).reshape(EPAD)
    zerosn = jnp.zeros((NPAD,), jnp.float32)

    val = _edge_vals(row3d, zerosn, col_p, w_p)

    x = inputs
    for W, b in ((W0, b0), (W1, b1), (W2, b2)):
        pf = _spmm(x.reshape(B * N * 2, FH), row3d, col_p, val)
        p0 = pf[0].reshape(B, NPAD, FH)[:, :N]
        p1 = pf[1].reshape(B, NPAD, FH)[:, :N]
        x = _dense(x, p0, p1, W, b.reshape(1, F))
    return _head(x, Wh1, bh1.reshape(1, F), Wh2, bh2.reshape(1, 32),
                 Wh3, bh3.reshape(1, 10))
